# Initial kernel scaffold; baseline (speedup 1.0000x reference)
import jax
import jax.numpy as jnp
from jax.experimental import pallas as pl


def kernel(x, W, b):
    raise NotImplementedError("write your pallas kernel here")


# speedup vs baseline: 5.5966x; 5.5966x over previous
import jax
import jax.numpy as jnp
from jax.experimental import pallas as pl

D_MODEL = 768
NUM_HEADS = 12
HEAD_DIM = 64
SCALE = HEAD_DIM ** -0.5
B = 2
C = 4096
N = B * C
K_CH = 409
K_PAD = 512
TILE = 256
I_CHUNK = 512
LANE_CHUNK = 1024


def _stage1_kernel(x_ref, w_ref, b_ref, v_ref, tks_ref, tki_ref):
    xt = x_ref[...]
    qkv = jnp.dot(xt, w_ref[...], preferred_element_type=jnp.float32)
    qkv = qkv + b_ref[...]
    q3 = qkv[:, 0:D_MODEL].reshape(TILE, NUM_HEADS, HEAD_DIM)
    k3 = qkv[:, D_MODEL:2 * D_MODEL].reshape(TILE, NUM_HEADS, HEAD_DIM)
    s = jax.lax.dot_general(
        q3, k3, (((2,), (2,)), ((0,), (0,))),
        preferred_element_type=jnp.float32) * SCALE
    m = jnp.max(s, axis=-1)
    iota = jax.lax.broadcasted_iota(jnp.int32, (TILE, NUM_HEADS, NUM_HEADS), 2)
    am = jnp.min(jnp.where(s == m[..., None], iota, NUM_HEADS), axis=-1)
    v_ref[...] = qkv[:, 2 * D_MODEL:3 * D_MODEL]
    tks_ref[...] = m
    tki_ref[...] = am


def _stage2_kernel(nrow_ref, ncol_ref, out_ref, feat_ref, ch_ref):
    c = pl.program_id(1)
    nr = nrow_ref[0]
    nc = ncol_ref[0]
    irow = (jax.lax.broadcasted_iota(jnp.int32, (1, LANE_CHUNK), 1)
            + c * LANE_CHUNK)
    rank = jnp.zeros((1, LANE_CHUNK), jnp.int32)
    for cm in range(C // I_CHUNK):
        vm = nc[cm * I_CHUNK:(cm + 1) * I_CHUNK, :]
        im = jax.lax.broadcasted_iota(
            jnp.int32, (I_CHUNK, 1), 0) + cm * I_CHUNK
        beats = (vm > nr) | ((vm == nr) & (im < irow))
        rank = rank + jnp.sum(
            jnp.where(beats, 1, 0), axis=0, keepdims=True)
    r_iota = jax.lax.broadcasted_iota(jnp.int32, (K_PAD, 1), 0)
    onehot = jnp.where(rank == r_iota, 1.0, 0.0)
    feat_c = jax.lax.dot_general(
        onehot, out_ref[0], (((1,), (0,)), ((), ())),
        precision=jax.lax.Precision.HIGHEST,
        preferred_element_type=jnp.float32)
    ch_c = jnp.sum(onehot * irow.astype(jnp.float32),
                   axis=1, keepdims=True)

    @pl.when(c == 0)
    def _init():
        feat_ref[0] = feat_c
        ch_ref[0] = ch_c

    @pl.when(c != 0)
    def _acc():
        feat_ref[0] += feat_c
        ch_ref[0] += ch_c


def kernel(x, W, b):
    x2 = x.reshape(N, D_MODEL)
    b2 = b.reshape(1, 3 * D_MODEL)

    vout, tks, tki = pl.pallas_call(
        _stage1_kernel,
        grid=(N // TILE,),
        in_specs=[
            pl.BlockSpec((TILE, D_MODEL), lambda i: (i, 0)),
            pl.BlockSpec((D_MODEL, 3 * D_MODEL), lambda i: (0, 0)),
            pl.BlockSpec((1, 3 * D_MODEL), lambda i: (0, 0)),
        ],
        out_specs=[
            pl.BlockSpec((TILE, D_MODEL), lambda i: (i, 0)),
            pl.BlockSpec((TILE, NUM_HEADS), lambda i: (i, 0)),
            pl.BlockSpec((TILE, NUM_HEADS), lambda i: (i, 0)),
        ],
        out_shape=[
            jax.ShapeDtypeStruct((N, D_MODEL), jnp.float32),
            jax.ShapeDtypeStruct((N, NUM_HEADS), jnp.float32),
            jax.ShapeDtypeStruct((N, NUM_HEADS), jnp.int32),
        ],
    )(x2, W, b2)

    v3 = vout.reshape(N, NUM_HEADS, HEAD_DIM)
    topk_scores = tks[..., None]
    topk_indices = tki[..., None]
    soft = jax.nn.softmax(topk_scores, axis=-1)
    rows = jnp.arange(N)[:, None, None]
    heads = jnp.arange(NUM_HEADS)[None, :, None]
    sparse_attn = (jnp.zeros((N, NUM_HEADS, NUM_HEADS), jnp.float32)
                   .at[rows, heads, topk_indices].set(soft))
    out3 = jnp.matmul(sparse_attn, v3)
    out = jnp.swapaxes(out3, 1, 2).reshape(B, C, D_MODEL)
    channel_importance = jnp.linalg.norm(out, axis=-1)

    nrow = channel_importance.reshape(B, 1, C)
    ncol = channel_importance.reshape(B, C, 1)

    featp, chp = pl.pallas_call(
        _stage2_kernel,
        grid=(B, C // LANE_CHUNK),
        in_specs=[
            pl.BlockSpec((1, 1, LANE_CHUNK), lambda i, c: (i, 0, c)),
            pl.BlockSpec((1, C, 1), lambda i, c: (i, 0, 0)),
            pl.BlockSpec((1, LANE_CHUNK, D_MODEL), lambda i, c: (i, c, 0)),
        ],
        out_specs=[
            pl.BlockSpec((1, K_PAD, D_MODEL), lambda i, c: (i, 0, 0)),
            pl.BlockSpec((1, K_PAD, 1), lambda i, c: (i, 0, 0)),
        ],
        out_shape=[
            jax.ShapeDtypeStruct((B, K_PAD, D_MODEL), jnp.float32),
            jax.ShapeDtypeStruct((B, K_PAD, 1), jnp.float32),
        ],
    )(nrow, ncol, out)

    sparse_feat = featp[:, :K_CH, :]
    ch_idx = chp[:, :K_CH, 0].astype(jnp.int32)
    return (sparse_feat, ch_idx, K_CH)


# speedup vs baseline: 6.3868x; 1.1412x over previous
import functools

import jax
import jax.numpy as jnp
from jax import lax
from jax.experimental import pallas as pl
from jax.experimental.pallas import tpu as pltpu, tpu_sc as plsc

D_MODEL = 768
NUM_HEADS = 12
HEAD_DIM = 64
SCALE = HEAD_DIM ** -0.5
B = 2
C = 4096
N = B * C
K_CH = 409
K_PAD = 512
TILE = 256
I_CHUNK = 512
LANE_CHUNK = 1024


def _stage1_kernel(x_ref, w_ref, b_ref, v_ref, tks_ref, tki_ref):
    xt = x_ref[...]
    qkv = jnp.dot(xt, w_ref[...], preferred_element_type=jnp.float32)
    qkv = qkv + b_ref[...]
    q3 = qkv[:, 0:D_MODEL].reshape(TILE, NUM_HEADS, HEAD_DIM)
    k3 = qkv[:, D_MODEL:2 * D_MODEL].reshape(TILE, NUM_HEADS, HEAD_DIM)
    s = jax.lax.dot_general(
        q3, k3, (((2,), (2,)), ((0,), (0,))),
        preferred_element_type=jnp.float32) * SCALE
    m = jnp.max(s, axis=-1)
    iota = jax.lax.broadcasted_iota(jnp.int32, (TILE, NUM_HEADS, NUM_HEADS), 2)
    am = jnp.min(jnp.where(s == m[..., None], iota, NUM_HEADS), axis=-1)
    v_ref[...] = qkv[:, 2 * D_MODEL:3 * D_MODEL]
    tks_ref[...] = m
    tki_ref[...] = am


def _stage2_kernel(nrow_ref, ncol_ref, ch_ref):
    c = pl.program_id(1)
    nr = nrow_ref[0]
    nc = ncol_ref[0]
    irow = (jax.lax.broadcasted_iota(jnp.int32, (1, LANE_CHUNK), 1)
            + c * LANE_CHUNK)
    rank = jnp.zeros((1, LANE_CHUNK), jnp.int32)
    for cm in range(C // I_CHUNK):
        vm = nc[cm * I_CHUNK:(cm + 1) * I_CHUNK, :]
        im = jax.lax.broadcasted_iota(
            jnp.int32, (I_CHUNK, 1), 0) + cm * I_CHUNK
        beats = (vm > nr) | ((vm == nr) & (im < irow))
        rank = rank + jnp.sum(
            jnp.where(beats, 1, 0), axis=0, keepdims=True)
    r_iota = jax.lax.broadcasted_iota(jnp.int32, (K_PAD, 1), 0)
    onehot = jnp.where(rank == r_iota, 1.0, 0.0)
    ch_c = jnp.sum(onehot * irow.astype(jnp.float32),
                   axis=1, keepdims=True)

    @pl.when(c == 0)
    def _init():
        ch_ref[0] = ch_c

    @pl.when(c != 0)
    def _acc():
        ch_ref[0] += ch_c


_SC_INFO = plsc.get_sparse_core_info()
_NW = _SC_INFO.num_cores * _SC_INFO.num_subcores
_B_PER_W = (B * K_PAD) // _NW


@functools.partial(
    pl.kernel,
    mesh=plsc.VectorSubcoreMesh(core_axis_name="c", subcore_axis_name="s"),
    out_type=jax.ShapeDtypeStruct((B * K_PAD, D_MODEL), jnp.float32),
    scratch_types=[
        pltpu.VMEM((_B_PER_W,), jnp.int32),
        pltpu.VMEM((_B_PER_W, D_MODEL), jnp.float32),
        pltpu.SemaphoreType.DMA,
    ],
)
def _sc_gather(table_hbm, idx_hbm, out_hbm, idx_v, rows_v, sem):
    wid = lax.axis_index("s") * _SC_INFO.num_cores + lax.axis_index("c")
    base = wid * _B_PER_W
    pltpu.sync_copy(idx_hbm.at[pl.ds(base, _B_PER_W)], idx_v)
    pltpu.async_copy(table_hbm.at[idx_v], rows_v, sem).wait()
    pltpu.sync_copy(rows_v, out_hbm.at[pl.ds(base, _B_PER_W)])


def kernel(x, W, b):
    x2 = x.reshape(N, D_MODEL)
    b2 = b.reshape(1, 3 * D_MODEL)

    vout, tks, tki = pl.pallas_call(
        _stage1_kernel,
        grid=(N // TILE,),
        in_specs=[
            pl.BlockSpec((TILE, D_MODEL), lambda i: (i, 0)),
            pl.BlockSpec((D_MODEL, 3 * D_MODEL), lambda i: (0, 0)),
            pl.BlockSpec((1, 3 * D_MODEL), lambda i: (0, 0)),
        ],
        out_specs=[
            pl.BlockSpec((TILE, D_MODEL), lambda i: (i, 0)),
            pl.BlockSpec((TILE, NUM_HEADS), lambda i: (i, 0)),
            pl.BlockSpec((TILE, NUM_HEADS), lambda i: (i, 0)),
        ],
        out_shape=[
            jax.ShapeDtypeStruct((N, D_MODEL), jnp.float32),
            jax.ShapeDtypeStruct((N, NUM_HEADS), jnp.float32),
            jax.ShapeDtypeStruct((N, NUM_HEADS), jnp.int32),
        ],
    )(x2, W, b2)

    v3 = vout.reshape(N, NUM_HEADS, HEAD_DIM)
    topk_scores = tks[..., None]
    topk_indices = tki[..., None]
    soft = jax.nn.softmax(topk_scores, axis=-1)
    rows = jnp.arange(N)[:, None, None]
    heads = jnp.arange(NUM_HEADS)[None, :, None]
    sparse_attn = (jnp.zeros((N, NUM_HEADS, NUM_HEADS), jnp.float32)
                   .at[rows, heads, topk_indices].set(soft))
    out3 = jnp.matmul(sparse_attn, v3)
    out = jnp.swapaxes(out3, 1, 2).reshape(B, C, D_MODEL)
    channel_importance = jnp.linalg.norm(out, axis=-1)

    nrow = channel_importance.reshape(B, 1, C)
    ncol = channel_importance.reshape(B, C, 1)

    chp = pl.pallas_call(
        _stage2_kernel,
        grid=(B, C // LANE_CHUNK),
        in_specs=[
            pl.BlockSpec((1, 1, LANE_CHUNK), lambda i, c: (i, 0, c)),
            pl.BlockSpec((1, C, 1), lambda i, c: (i, 0, 0)),
        ],
        out_specs=pl.BlockSpec((1, K_PAD, 1), lambda i, c: (i, 0, 0)),
        out_shape=jax.ShapeDtypeStruct((B, K_PAD, 1), jnp.float32),
    )(nrow, ncol)

    ch_all = chp[:, :, 0].astype(jnp.int32)
    gidx = (ch_all + jnp.arange(B, dtype=jnp.int32)[:, None] * C).reshape(-1)
    rows = _sc_gather(out.reshape(N, D_MODEL), gidx)

    sparse_feat = rows.reshape(B, K_PAD, D_MODEL)[:, :K_CH, :]
    ch_idx = ch_all[:, :K_CH]
    return (sparse_feat, ch_idx, K_CH)
